# baseline (speedup 1.0000x reference)
import numpy as np
import jax
import jax.numpy as jnp
from jax.experimental import pallas as pl
from jax.experimental.pallas import tpu as pltpu

B, C, L, D = 4, 128, 512, 16
DM, DFF, H = 512, 2048, 8
LD = L * D
DH = DM // H
NK = 4
KC = LD // NK
_SCALE = 1.0 / float(np.sqrt(DH))
_BF = jnp.bfloat16
_F32 = jnp.float32


def _pos_embed_np():
    pe = np.zeros((C, DM), dtype=np.float32)
    position = np.arange(0, C, dtype=np.float32)[:, None]
    div_term = np.exp(np.arange(0, DM, 2, dtype=np.float32) * -(np.log(10000.0) / DM))
    pe[:, 0::2] = np.sin(position * div_term)
    pe[:, 1::2] = np.cos(position * div_term)
    return pe


def _ln(x, g, b):
    mu = jnp.mean(x, axis=-1, keepdims=True)
    xc = x - mu
    var = jnp.mean(xc * xc, axis=-1, keepdims=True)
    return xc * jax.lax.rsqrt(var + 1e-5) * g + b


def _dot(a, b):
    return jax.lax.dot_general(a, b, (((1,), (0,)), ((), ())),
                               preferred_element_type=_F32)


def _encoder_body(x_ref, wd_ref, bd_ref, pe_ref, wq_ref, bq_ref, wk_ref, bk_ref,
                  wv_ref, bv_ref, wo_ref, bo_ref, g1_ref, be1_ref, w1_ref,
                  bf1_ref, w2_ref, bf2_ref, g2_ref, be2_ref, o_ref):
    k = pl.program_id(0)
    b = pl.program_id(1)
    part = _dot(x_ref[0].astype(_BF), wd_ref[...])

    @pl.when(k == 0)
    def _init():
        o_ref[b] = part

    @pl.when(k > 0)
    def _accum():
        o_ref[b] += part

    @pl.when(k == NK - 1)
    def _finalize():
        h = o_ref[b] + bd_ref[...] + pe_ref[...]
        res = h
        hb = h.astype(_BF)
        q = _dot(hb, wq_ref[...]) + bq_ref[...]
        kk = _dot(hb, wk_ref[...]) + bk_ref[...]
        v = _dot(hb, wv_ref[...]) + bv_ref[...]
        outs = []
        for i in range(H):
            qh = q[:, i * DH:(i + 1) * DH].astype(_BF)
            kh = kk[:, i * DH:(i + 1) * DH].astype(_BF)
            vh = v[:, i * DH:(i + 1) * DH].astype(_BF)
            s = jax.lax.dot_general(qh, kh, (((1,), (1,)), ((), ())),
                                    preferred_element_type=_F32) * _SCALE
            s = s - jnp.max(s, axis=-1, keepdims=True)
            e = jnp.exp(s)
            a = e / jnp.sum(e, axis=-1, keepdims=True)
            outs.append(_dot(a.astype(_BF), vh))
        o = jnp.concatenate(outs, axis=1)
        o = _dot(o.astype(_BF), wo_ref[...]) + bo_ref[...]
        h = _ln(res + o, g1_ref[...], be1_ref[...])
        res = h
        m = _dot(h.astype(_BF), w1_ref[...]) + bf1_ref[...]
        m = jnp.maximum(m, 0.0)
        m = _dot(m.astype(_BF), w2_ref[...]) + bf2_ref[...]
        o_ref[b] = _ln(res + m, g2_ref[...], be2_ref[...])


def kernel(x, Wd, bd, Wq, bq, Wk, bk, Wv, bv, Wo, bo, g1, be1, W1, bf1, W2, bf2, g2, be2):
    xf = x.reshape(B, C, LD)
    pe = jnp.asarray(_pos_embed_np())
    Wd_bf, Wq_bf, Wk_bf, Wv_bf, Wo_bf, W1_bf, W2_bf = (
        w.astype(_BF) for w in (Wd, Wq, Wk, Wv, Wo, W1, W2))

    def row(a, n):
        return a.reshape(1, n)

    full = lambda shape: pl.BlockSpec(shape, lambda k, b: (0,) * len(shape))
    out = pl.pallas_call(
        _encoder_body,
        grid=(NK, B),
        in_specs=[
            pl.BlockSpec((1, C, KC), lambda k, b: (b, 0, k)),
            pl.BlockSpec((KC, DM), lambda k, b: (k, 0)),
            full((1, DM)),
            full((C, DM)),
            full((DM, DM)),
            full((1, DM)),
            full((DM, DM)),
            full((1, DM)),
            full((DM, DM)),
            full((1, DM)),
            full((DM, DM)),
            full((1, DM)),
            full((1, DM)),
            full((1, DM)),
            full((DM, DFF)),
            full((1, DFF)),
            full((DFF, DM)),
            full((1, DM)),
            full((1, DM)),
            full((1, DM)),
        ],
        out_specs=pl.BlockSpec((B, C, DM), lambda k, b: (0, 0, 0)),
        out_shape=jax.ShapeDtypeStruct((B, C, DM), _F32),
        compiler_params=pltpu.CompilerParams(
            vmem_limit_bytes=60 * 1024 * 1024),
    )(xf, Wd_bf, row(bd, DM), pe, Wq_bf, row(bq, DM), Wk_bf, row(bk, DM),
      Wv_bf, row(bv, DM), Wo_bf, row(bo, DM), row(g1, DM), row(be1, DM),
      W1_bf, row(bf1, DFF), W2_bf, row(bf2, DM), row(g2, DM), row(be2, DM))
    return out
